# baseline (speedup 1.0000x reference)
import jax
import jax.numpy as jnp
from jax import lax
from jax.experimental import pallas as pl
from jax.experimental.pallas import tpu as pltpu
from jax.experimental.pallas import tpu_sc as plsc

N_NODES = 10000
N_EDGES = 320000
ALPHA = 0.8

_NC = 2
_NS = 16
_CHUNK = 128
_E_PAD = 327680
_NCHUNKS = _E_PAD // _CHUNK
_NCH_IN = N_EDGES // _CHUNK
_ROWS_PAD = 10240
_WR = _ROWS_PAD // _NS
_KB = 2

_mesh = plsc.VectorSubcoreMesh(core_axis_name="c", subcore_axis_name="s")


def _make_segsum():
    n_chunks = _NCHUNKS // _NS

    def body(zs, e2, zeros, out, srcb, dstb, rows, acc, gsem, ssem):
        c = lax.axis_index("c")
        s = lax.axis_index("s")
        pltpu.sync_copy(zeros, acc.at[pl.ds(s * _WR, _WR)])
        plsc.subcore_barrier()

        def edge_sweep(zp):
            def chunk_body(j, carry):
                b = j % 2
                base = s * n_chunks + j

                @pl.when(j >= 2)
                def _():
                    pltpu.make_async_copy(
                        rows.at[pl.ds(b * _CHUNK, _CHUNK)],
                        acc.at[dstb.at[b]], ssem).wait()
                pltpu.sync_copy(e2.at[0, pl.ds(base, 1)], srcb)
                pltpu.sync_copy(e2.at[1, pl.ds(base, 1)],
                                dstb.at[pl.ds(b, 1)])
                pltpu.async_copy(zp.at[srcb.at[0]],
                                 rows.at[pl.ds(b * _CHUNK, _CHUNK)],
                                 gsem).wait()
                pltpu.async_copy(rows.at[pl.ds(b * _CHUNK, _CHUNK)],
                                 acc.at[dstb.at[b]], ssem, add=True)
                return carry
            lax.fori_loop(0, n_chunks, chunk_body, 0)
            for b in range(2):
                pltpu.make_async_copy(
                    rows.at[pl.ds(b * _CHUNK, _CHUNK)],
                    acc.at[dstb.at[b]], ssem).wait()

        @pl.when(c == 0)
        def _():
            edge_sweep(zs.at[0])

        @pl.when(c == 1)
        def _():
            edge_sweep(zs.at[1])

        plsc.subcore_barrier()
        pltpu.sync_copy(acc.at[pl.ds(s * _WR, _WR)],
                        out.at[c, pl.ds(s * _WR, _WR)])

    return pl.kernel(
        body,
        mesh=_mesh,
        out_type=jax.ShapeDtypeStruct((_NC, _ROWS_PAD, 128), jnp.float32),
        scratch_types=[
            pltpu.VMEM((1, _CHUNK), jnp.int32),
            pltpu.VMEM((2, _CHUNK), jnp.int32),
            pltpu.VMEM((2 * _CHUNK, 128), jnp.float32),
            pltpu.VMEM_SHARED((_ROWS_PAD, 128), jnp.float32),
            pltpu.SemaphoreType.DMA,
            pltpu.SemaphoreType.DMA,
        ],
    )


_segsum = _make_segsum()

_R = 1000
_G = N_NODES // _R


def _elu(v):
    return jnp.where(v > 0, v, jnp.exp(v) - 1.0)


def _mm_t(a, w):
    return lax.dot_general(a, w, (((1,), (1,)), ((), ())),
                           preferred_element_type=jnp.float32)


def _dis_of(cnt_ref):
    return lax.rsqrt(1.0 + cnt_ref[...])


def _edges_prep(ei3):
    blk = 128

    def body(e_ref, o_ref):
        i = pl.program_id(1)
        row = i * blk + lax.broadcasted_iota(jnp.int32, (blk, _CHUNK), 0)
        o_ref[...] = jnp.where(row < _NCH_IN, e_ref[...],
                               jnp.int32(N_NODES))

    return pl.pallas_call(
        body,
        grid=(2, _NCHUNKS // blk),
        in_specs=[pl.BlockSpec((1, blk, _CHUNK), lambda p, i: (p, i, 0))],
        out_specs=pl.BlockSpec((1, blk, _CHUNK), lambda p, i: (p, i, 0)),
        out_shape=jax.ShapeDtypeStruct((2, _NCHUNKS, _CHUNK), jnp.int32),
    )(ei3)


def _pad256(x):
    def body(x_ref, o_ref):
        o_ref[...] = jnp.concatenate(
            [x_ref[...], jnp.zeros((_R, 128), jnp.float32)], axis=1)

    return pl.pallas_call(
        body,
        grid=(_G,),
        in_specs=[pl.BlockSpec((_R, 128), lambda i: (i, 0))],
        out_specs=pl.BlockSpec((_R, 256), lambda i: (i, 0)),
        out_shape=jax.ShapeDtypeStruct((N_NODES, 256), jnp.float32),
    )(x)


def _take_left(h):
    def body(h_ref, o_ref):
        o_ref[...] = h_ref[:, :128]

    return pl.pallas_call(
        body,
        grid=(_G,),
        in_specs=[pl.BlockSpec((_R, 256), lambda i: (i, 0))],
        out_specs=pl.BlockSpec((_R, 128), lambda i: (i, 0)),
        out_shape=jax.ShapeDtypeStruct((N_NODES, 128), jnp.float32),
    )(h)


def _zscale(h, cnt):
    def body(h_ref, cnt_ref, z_ref):
        z = h_ref[...] * _dis_of(cnt_ref)
        z_ref[0] = z[:, :128]
        z_ref[1] = z[:, 128:]

    return pl.pallas_call(
        body,
        grid=(_G,),
        in_specs=[pl.BlockSpec((_R, 256), lambda i: (i, 0)),
                  pl.BlockSpec((_R, 1), lambda i: (i, 0))],
        out_specs=pl.BlockSpec((2, _R, 128), lambda i: (0, i, 0)),
        out_shape=jax.ShapeDtypeStruct((2, _ROWS_PAD, 128), jnp.float32),
    )(h, cnt)


def _combine(h, cnt, acc, w, b, ef):
    def body(h_ref, cnt_ref, a_ref, w_ref, b_ref, ef_ref, o_ref):
        dis = _dis_of(cnt_ref)
        hb = h_ref[...]
        acc2 = jnp.concatenate([a_ref[0], a_ref[1]], axis=1)
        agg = dis * (acc2 + dis * hb)
        hpre = ALPHA * hb + (1.0 - ALPHA) * agg
        v = _mm_t(hpre, w_ref[...]) + b_ref[...]
        o_ref[...] = jnp.where(ef_ref[...] > 0.5, _elu(v), v)

    return pl.pallas_call(
        body,
        grid=(_G,),
        in_specs=[pl.BlockSpec((_R, 256), lambda i: (i, 0)),
                  pl.BlockSpec((_R, 1), lambda i: (i, 0)),
                  pl.BlockSpec((2, _R, 128), lambda i: (0, i, 0)),
                  pl.BlockSpec((256, 256), lambda i: (0, 0)),
                  pl.BlockSpec((1, 256), lambda i: (0, 0)),
                  pl.BlockSpec((1, 1), lambda i: (0, 0))],
        out_specs=pl.BlockSpec((_R, 256), lambda i: (i, 0)),
        out_shape=jax.ShapeDtypeStruct((N_NODES, 256), jnp.float32),
    )(h, cnt, acc, w, b, ef)


def kernel(x, edge_index, W1, b1, W2, b2, W3, b3):
    ei3 = edge_index.astype(jnp.int32).reshape(2, _NCH_IN, _CHUNK)
    e2 = _edges_prep(ei3)
    zeros_blk = jnp.zeros((_WR, 128), jnp.float32)
    ones_tab = jnp.ones((2, _ROWS_PAD, 128), jnp.float32)

    ws = jnp.stack([jnp.pad(W1, ((0, 0), (0, 128))), W2,
                    jnp.pad(W3, ((0, 128), (0, 0)))])
    bs = jnp.stack([b1, b2, jnp.pad(b3, (0, 128))]).reshape(3, 1, 256)
    efs = jnp.array([1.0, 1.0, 0.0], jnp.float32).reshape(3, 1, 1)

    cnt = _segsum(ones_tab, e2, zeros_blk)[0][:, :1]

    def layer(i, h):
        w = lax.dynamic_index_in_dim(ws, i, 0, keepdims=False)
        b = lax.dynamic_index_in_dim(bs, i, 0, keepdims=False)
        ef = lax.dynamic_index_in_dim(efs, i, 0, keepdims=False)
        zs = _zscale(h, cnt)
        acc = _segsum(zs, e2, zeros_blk)
        return _combine(h, cnt, acc, w, b, ef)

    h0 = _pad256(x)
    n_layers = 3 + (ei3[0, 0, 0] - ei3[0, 0, 0])
    h3 = lax.fori_loop(0, n_layers, layer, h0)
    return _take_left(h3)


# speedup vs baseline: 1.1580x; 1.1580x over previous
import jax
import jax.numpy as jnp
from jax import lax
from jax.experimental import pallas as pl
from jax.experimental.pallas import tpu as pltpu
from jax.experimental.pallas import tpu_sc as plsc

N_NODES = 10000
N_EDGES = 320000
ALPHA = 0.8

_NC = 2
_NS = 16
_CHUNK = 128
_E_PAD = 327680
_NCHUNKS = _E_PAD // _CHUNK
_NCH_IN = N_EDGES // _CHUNK
_ROWS_PAD = 10240
_WR = _ROWS_PAD // _NS
_GRP = 16

_mesh = plsc.VectorSubcoreMesh(core_axis_name="c", subcore_axis_name="s")


def _make_segsum():
    n_chunks = _NCHUNKS // _NS

    def body(zs, e2, zeros, out, srcb, dstb, rows, acc, gsem, ssem):
        c = lax.axis_index("c")
        s = lax.axis_index("s")
        pltpu.sync_copy(zeros, acc.at[pl.ds(s * _WR, _WR)])
        plsc.subcore_barrier()

        def edge_sweep(zp):
            def group_body(g, carry):
                base = s * n_chunks + g * _GRP
                pltpu.sync_copy(e2.at[0, pl.ds(base, _GRP)], srcb)
                pltpu.sync_copy(e2.at[1, pl.ds(base, _GRP)], dstb)
                sh = []
                for jj in range(_GRP):
                    b = jj % 2
                    if jj >= 2:
                        sh[jj - 2].wait()
                    pltpu.async_copy(
                        zp.at[srcb.at[jj]],
                        rows.at[pl.ds(b * _CHUNK, _CHUNK)], gsem).wait()
                    sh.append(pltpu.async_copy(
                        rows.at[pl.ds(b * _CHUNK, _CHUNK)],
                        acc.at[dstb.at[jj]], ssem, add=True))
                sh[_GRP - 2].wait()
                sh[_GRP - 1].wait()
                return carry
            lax.fori_loop(0, n_chunks // _GRP, group_body, 0)

        @pl.when(c == 0)
        def _():
            edge_sweep(zs.at[0])

        @pl.when(c == 1)
        def _():
            edge_sweep(zs.at[1])

        plsc.subcore_barrier()
        pltpu.sync_copy(acc.at[pl.ds(s * _WR, _WR)],
                        out.at[c, pl.ds(s * _WR, _WR)])

    return pl.kernel(
        body,
        mesh=_mesh,
        out_type=jax.ShapeDtypeStruct((_NC, _ROWS_PAD, 128), jnp.float32),
        scratch_types=[
            pltpu.VMEM((_GRP, _CHUNK), jnp.int32),
            pltpu.VMEM((_GRP, _CHUNK), jnp.int32),
            pltpu.VMEM((2 * _CHUNK, 128), jnp.float32),
            pltpu.VMEM_SHARED((_ROWS_PAD, 128), jnp.float32),
            pltpu.SemaphoreType.DMA,
            pltpu.SemaphoreType.DMA,
        ],
    )


_segsum = _make_segsum()

_R = 1000
_G = N_NODES // _R


def _elu(v):
    return jnp.where(v > 0, v, jnp.exp(v) - 1.0)


def _mm_t(a, w):
    return lax.dot_general(a, w, (((1,), (1,)), ((), ())),
                           preferred_element_type=jnp.float32)


def _dis_of(cnt_ref):
    return lax.rsqrt(1.0 + cnt_ref[...])


def _edges_prep(ei3):
    blk = 128

    def body(e_ref, o_ref):
        i = pl.program_id(1)
        row = i * blk + lax.broadcasted_iota(jnp.int32, (blk, _CHUNK), 0)
        o_ref[...] = jnp.where(row < _NCH_IN, e_ref[...],
                               jnp.int32(N_NODES))

    return pl.pallas_call(
        body,
        grid=(2, _NCHUNKS // blk),
        in_specs=[pl.BlockSpec((1, blk, _CHUNK), lambda p, i: (p, i, 0))],
        out_specs=pl.BlockSpec((1, blk, _CHUNK), lambda p, i: (p, i, 0)),
        out_shape=jax.ShapeDtypeStruct((2, _NCHUNKS, _CHUNK), jnp.int32),
    )(ei3)


def _pad256(x):
    def body(x_ref, o_ref):
        o_ref[...] = jnp.concatenate(
            [x_ref[...], jnp.zeros((_R, 128), jnp.float32)], axis=1)

    return pl.pallas_call(
        body,
        grid=(_G,),
        in_specs=[pl.BlockSpec((_R, 128), lambda i: (i, 0))],
        out_specs=pl.BlockSpec((_R, 256), lambda i: (i, 0)),
        out_shape=jax.ShapeDtypeStruct((N_NODES, 256), jnp.float32),
    )(x)


def _take_left(h):
    def body(h_ref, o_ref):
        o_ref[...] = h_ref[:, :128]

    return pl.pallas_call(
        body,
        grid=(_G,),
        in_specs=[pl.BlockSpec((_R, 256), lambda i: (i, 0))],
        out_specs=pl.BlockSpec((_R, 128), lambda i: (i, 0)),
        out_shape=jax.ShapeDtypeStruct((N_NODES, 128), jnp.float32),
    )(h)


def _zscale(h, cnt):
    def body(h_ref, cnt_ref, z_ref):
        z = h_ref[...] * _dis_of(cnt_ref)
        z_ref[0] = z[:, :128]
        z_ref[1] = z[:, 128:]

    return pl.pallas_call(
        body,
        grid=(_G,),
        in_specs=[pl.BlockSpec((_R, 256), lambda i: (i, 0)),
                  pl.BlockSpec((_R, 1), lambda i: (i, 0))],
        out_specs=pl.BlockSpec((2, _R, 128), lambda i: (0, i, 0)),
        out_shape=jax.ShapeDtypeStruct((2, _ROWS_PAD, 128), jnp.float32),
    )(h, cnt)


def _combine(h, cnt, acc, w, b, ef):
    def body(h_ref, cnt_ref, a_ref, w_ref, b_ref, ef_ref, o_ref):
        dis = _dis_of(cnt_ref)
        hb = h_ref[...]
        acc2 = jnp.concatenate([a_ref[0], a_ref[1]], axis=1)
        agg = dis * (acc2 + dis * hb)
        hpre = ALPHA * hb + (1.0 - ALPHA) * agg
        v = _mm_t(hpre, w_ref[...]) + b_ref[...]
        o_ref[...] = jnp.where(ef_ref[...] > 0.5, _elu(v), v)

    return pl.pallas_call(
        body,
        grid=(_G,),
        in_specs=[pl.BlockSpec((_R, 256), lambda i: (i, 0)),
                  pl.BlockSpec((_R, 1), lambda i: (i, 0)),
                  pl.BlockSpec((2, _R, 128), lambda i: (0, i, 0)),
                  pl.BlockSpec((256, 256), lambda i: (0, 0)),
                  pl.BlockSpec((1, 256), lambda i: (0, 0)),
                  pl.BlockSpec((1, 1), lambda i: (0, 0))],
        out_specs=pl.BlockSpec((_R, 256), lambda i: (i, 0)),
        out_shape=jax.ShapeDtypeStruct((N_NODES, 256), jnp.float32),
    )(h, cnt, acc, w, b, ef)


def kernel(x, edge_index, W1, b1, W2, b2, W3, b3):
    ei3 = edge_index.astype(jnp.int32).reshape(2, _NCH_IN, _CHUNK)
    e2 = _edges_prep(ei3)
    zeros_blk = jnp.zeros((_WR, 128), jnp.float32)
    ones_tab = jnp.ones((2, _ROWS_PAD, 128), jnp.float32)

    ws = jnp.stack([jnp.pad(W1, ((0, 0), (0, 128))), W2,
                    jnp.pad(W3, ((0, 128), (0, 0)))])
    bs = jnp.stack([b1, b2, jnp.pad(b3, (0, 128))]).reshape(3, 1, 256)
    efs = jnp.array([1.0, 1.0, 0.0], jnp.float32).reshape(3, 1, 1)

    cnt = _segsum(ones_tab, e2, zeros_blk)[0][:, :1]

    def layer(i, h):
        w = lax.dynamic_index_in_dim(ws, i, 0, keepdims=False)
        b = lax.dynamic_index_in_dim(bs, i, 0, keepdims=False)
        ef = lax.dynamic_index_in_dim(efs, i, 0, keepdims=False)
        zs = _zscale(h, cnt)
        acc = _segsum(zs, e2, zeros_blk)
        return _combine(h, cnt, acc, w, b, ef)

    h0 = _pad256(x)
    n_layers = 3 + (ei3[0, 0, 0] - ei3[0, 0, 0])
    h3 = lax.fori_loop(0, n_layers, layer, h0)
    return _take_left(h3)


# speedup vs baseline: 1.4705x; 1.2699x over previous
import jax
import jax.numpy as jnp
from jax import lax
from jax.experimental import pallas as pl
from jax.experimental.pallas import tpu as pltpu
from jax.experimental.pallas import tpu_sc as plsc

N_NODES = 10000
N_EDGES = 320000
ALPHA = 0.8

_NC = 2
_NS = 16
_CHUNK = 128
_E_PAD = 327680
_NCHUNKS = _E_PAD // _CHUNK
_NCH_IN = N_EDGES // _CHUNK
_ROWS_PAD = 10240
_WR = _ROWS_PAD // _NS
_GRP = 16

_mesh = plsc.VectorSubcoreMesh(core_axis_name="c", subcore_axis_name="s")


def _make_segsum():
    n_chunks = _NCHUNKS // _NS

    def body(zs, e2, zeros, out, srcb, dstb, rows, acc, gsem, ssem):
        c = lax.axis_index("c")
        s = lax.axis_index("s")
        pltpu.sync_copy(zeros, acc.at[pl.ds(s * _WR, _WR)])
        plsc.subcore_barrier()

        def edge_sweep(zp):
            def group_body(g, carry):
                base = s * n_chunks + g * _GRP
                pltpu.sync_copy(e2.at[0, pl.ds(base, _GRP)], srcb)
                pltpu.sync_copy(e2.at[1, pl.ds(base, _GRP)], dstb)
                sh = []
                for jj in range(_GRP):
                    b = jj % 2
                    if jj >= 2:
                        sh[jj - 2].wait()
                    pltpu.async_copy(
                        zp.at[srcb.at[jj]],
                        rows.at[pl.ds(b * _CHUNK, _CHUNK)], gsem).wait()
                    sh.append(pltpu.async_copy(
                        rows.at[pl.ds(b * _CHUNK, _CHUNK)],
                        acc.at[dstb.at[jj]], ssem, add=True))
                sh[_GRP - 2].wait()
                sh[_GRP - 1].wait()
                return carry
            lax.fori_loop(0, n_chunks // _GRP, group_body, 0)

        @pl.when(c == 0)
        def _():
            edge_sweep(zs.at[0])

        @pl.when(c == 1)
        def _():
            edge_sweep(zs.at[1])

        plsc.subcore_barrier()
        pltpu.sync_copy(acc.at[pl.ds(s * _WR, _WR)],
                        out.at[c, pl.ds(s * _WR, _WR)])

    return pl.kernel(
        body,
        mesh=_mesh,
        out_type=jax.ShapeDtypeStruct((_NC, _ROWS_PAD, 128), jnp.float32),
        scratch_types=[
            pltpu.VMEM((_GRP, _CHUNK), jnp.int32),
            pltpu.VMEM((_GRP, _CHUNK), jnp.int32),
            pltpu.VMEM((2 * _CHUNK, 128), jnp.float32),
            pltpu.VMEM_SHARED((_ROWS_PAD, 128), jnp.float32),
            pltpu.SemaphoreType.DMA,
            pltpu.SemaphoreType.DMA,
        ],
    )


_segsum = _make_segsum()

def _make_degsum():
    n_chunks = _NCHUNKS // (_NC * _NS)
    grp = 16

    def body(e2, zeros, ones, out, dstb, rows1, acc, ssem):
        c = lax.axis_index("c")
        s = lax.axis_index("s")
        pltpu.sync_copy(zeros, acc.at[pl.ds(s * _WR, _WR)])
        pltpu.sync_copy(ones, rows1)
        plsc.subcore_barrier()

        def group_body(g, carry):
            base = c * (_NCHUNKS // 2) + s * n_chunks + g * grp
            pltpu.sync_copy(e2.at[1, pl.ds(base, grp)], dstb)
            sh = []
            for jj in range(grp):
                sh.append(pltpu.async_copy(
                    rows1, acc.at[dstb.at[jj]], ssem, add=True))
            for h in sh:
                h.wait()
            return carry
        lax.fori_loop(0, n_chunks // grp, group_body, 0)

        plsc.subcore_barrier()
        pltpu.sync_copy(acc.at[pl.ds(s * _WR, _WR)],
                        out.at[c, pl.ds(s * _WR, _WR)])

    return pl.kernel(
        body,
        mesh=_mesh,
        out_type=jax.ShapeDtypeStruct((_NC, _ROWS_PAD, 128), jnp.float32),
        scratch_types=[
            pltpu.VMEM((grp, _CHUNK), jnp.int32),
            pltpu.VMEM((_CHUNK, 128), jnp.float32),
            pltpu.VMEM_SHARED((_ROWS_PAD, 128), jnp.float32),
            pltpu.SemaphoreType.DMA,
        ],
    )


_degsum = _make_degsum()


_R = 1000
_G = N_NODES // _R


def _elu(v):
    return jnp.where(v > 0, v, jnp.exp(v) - 1.0)


def _mm_t(a, w):
    return lax.dot_general(a, w, (((1,), (1,)), ((), ())),
                           preferred_element_type=jnp.float32)


def _dis_of(cnt_ref):
    return lax.rsqrt(1.0 + cnt_ref[...])


def _edges_prep(ei3):
    blk = 128

    def body(e_ref, o_ref):
        i = pl.program_id(1)
        row = i * blk + lax.broadcasted_iota(jnp.int32, (blk, _CHUNK), 0)
        o_ref[...] = jnp.where(row < _NCH_IN, e_ref[...],
                               jnp.int32(N_NODES))

    return pl.pallas_call(
        body,
        grid=(2, _NCHUNKS // blk),
        in_specs=[pl.BlockSpec((1, blk, _CHUNK), lambda p, i: (p, i, 0))],
        out_specs=pl.BlockSpec((1, blk, _CHUNK), lambda p, i: (p, i, 0)),
        out_shape=jax.ShapeDtypeStruct((2, _NCHUNKS, _CHUNK), jnp.int32),
    )(ei3)


def _pad256(x):
    def body(x_ref, o_ref):
        o_ref[...] = jnp.concatenate(
            [x_ref[...], jnp.zeros((_R, 128), jnp.float32)], axis=1)

    return pl.pallas_call(
        body,
        grid=(_G,),
        in_specs=[pl.BlockSpec((_R, 128), lambda i: (i, 0))],
        out_specs=pl.BlockSpec((_R, 256), lambda i: (i, 0)),
        out_shape=jax.ShapeDtypeStruct((N_NODES, 256), jnp.float32),
    )(x)


def _take_left(h):
    def body(h_ref, o_ref):
        o_ref[...] = h_ref[:, :128]

    return pl.pallas_call(
        body,
        grid=(_G,),
        in_specs=[pl.BlockSpec((_R, 256), lambda i: (i, 0))],
        out_specs=pl.BlockSpec((_R, 128), lambda i: (i, 0)),
        out_shape=jax.ShapeDtypeStruct((N_NODES, 128), jnp.float32),
    )(h)


def _zscale(h, cnt):
    def body(h_ref, cnt_ref, z_ref):
        z = h_ref[...] * _dis_of(cnt_ref)
        z_ref[0] = z[:, :128]
        z_ref[1] = z[:, 128:]

    return pl.pallas_call(
        body,
        grid=(_G,),
        in_specs=[pl.BlockSpec((_R, 256), lambda i: (i, 0)),
                  pl.BlockSpec((_R, 1), lambda i: (i, 0))],
        out_specs=pl.BlockSpec((2, _R, 128), lambda i: (0, i, 0)),
        out_shape=jax.ShapeDtypeStruct((2, _ROWS_PAD, 128), jnp.float32),
    )(h, cnt)


def _combine(h, cnt, acc, w, b, ef):
    def body(h_ref, cnt_ref, a_ref, w_ref, b_ref, ef_ref, o_ref):
        dis = _dis_of(cnt_ref)
        hb = h_ref[...]
        acc2 = jnp.concatenate([a_ref[0], a_ref[1]], axis=1)
        agg = dis * (acc2 + dis * hb)
        hpre = ALPHA * hb + (1.0 - ALPHA) * agg
        v = _mm_t(hpre, w_ref[...]) + b_ref[...]
        o_ref[...] = jnp.where(ef_ref[...] > 0.5, _elu(v), v)

    return pl.pallas_call(
        body,
        grid=(_G,),
        in_specs=[pl.BlockSpec((_R, 256), lambda i: (i, 0)),
                  pl.BlockSpec((_R, 1), lambda i: (i, 0)),
                  pl.BlockSpec((2, _R, 128), lambda i: (0, i, 0)),
                  pl.BlockSpec((256, 256), lambda i: (0, 0)),
                  pl.BlockSpec((1, 256), lambda i: (0, 0)),
                  pl.BlockSpec((1, 1), lambda i: (0, 0))],
        out_specs=pl.BlockSpec((_R, 256), lambda i: (i, 0)),
        out_shape=jax.ShapeDtypeStruct((N_NODES, 256), jnp.float32),
    )(h, cnt, acc, w, b, ef)


def kernel(x, edge_index, W1, b1, W2, b2, W3, b3):
    ei3 = edge_index.astype(jnp.int32).reshape(2, _NCH_IN, _CHUNK)
    e2 = _edges_prep(ei3)
    zeros_blk = jnp.zeros((_WR, 128), jnp.float32)
    ones_sq = jnp.ones((_CHUNK, 128), jnp.float32)

    ws = jnp.stack([jnp.pad(W1, ((0, 0), (0, 128))), W2,
                    jnp.pad(W3, ((0, 128), (0, 0)))])
    bs = jnp.stack([b1, b2, jnp.pad(b3, (0, 128))]).reshape(3, 1, 256)
    efs = jnp.array([1.0, 1.0, 0.0], jnp.float32).reshape(3, 1, 1)

    acc_d = _degsum(e2, zeros_blk, ones_sq)
    cnt = acc_d[0][:, :1] + acc_d[1][:, :1]

    def layer(i, h):
        w = lax.dynamic_index_in_dim(ws, i, 0, keepdims=False)
        b = lax.dynamic_index_in_dim(bs, i, 0, keepdims=False)
        ef = lax.dynamic_index_in_dim(efs, i, 0, keepdims=False)
        zs = _zscale(h, cnt)
        acc = _segsum(zs, e2, zeros_blk)
        return _combine(h, cnt, acc, w, b, ef)

    h0 = _pad256(x)
    n_layers = 3 + (ei3[0, 0, 0] - ei3[0, 0, 0])
    h3 = lax.fori_loop(0, n_layers, layer, h0)
    return _take_left(h3)


# speedup vs baseline: 1.4850x; 1.0098x over previous
import jax
import jax.numpy as jnp
from jax import lax
from jax.experimental import pallas as pl
from jax.experimental.pallas import tpu as pltpu
from jax.experimental.pallas import tpu_sc as plsc

N_NODES = 10000
N_EDGES = 320000
ALPHA = 0.8

_NC = 2
_NS = 16
_CHUNK = 128
_E_PAD = 327680
_NCHUNKS = _E_PAD // _CHUNK
_NCH_IN = N_EDGES // _CHUNK
_ROWS_PAD = 10240
_WR = _ROWS_PAD // _NS
_GRP = 32

_mesh = plsc.VectorSubcoreMesh(core_axis_name="c", subcore_axis_name="s")


def _make_segsum():
    n_chunks = _NCHUNKS // _NS

    def body(zs, e2, zeros, out, srcb, dstb, rows, acc, gsem, ssem):
        c = lax.axis_index("c")
        s = lax.axis_index("s")
        pltpu.sync_copy(zeros, acc.at[pl.ds(s * _WR, _WR)])
        plsc.subcore_barrier()

        def edge_sweep(zp):
            def group_body(g, carry):
                base = s * n_chunks + g * _GRP
                pltpu.sync_copy(e2.at[0, pl.ds(base, _GRP)], srcb)
                pltpu.sync_copy(e2.at[1, pl.ds(base, _GRP)], dstb)
                sh = []
                for jj in range(_GRP):
                    b = jj % 2
                    if jj >= 2:
                        sh[jj - 2].wait()
                    pltpu.async_copy(
                        zp.at[srcb.at[jj]],
                        rows.at[pl.ds(b * _CHUNK, _CHUNK)], gsem).wait()
                    sh.append(pltpu.async_copy(
                        rows.at[pl.ds(b * _CHUNK, _CHUNK)],
                        acc.at[dstb.at[jj]], ssem, add=True))
                sh[_GRP - 2].wait()
                sh[_GRP - 1].wait()
                return carry
            lax.fori_loop(0, n_chunks // _GRP, group_body, 0)

        @pl.when(c == 0)
        def _():
            edge_sweep(zs.at[0])

        @pl.when(c == 1)
        def _():
            edge_sweep(zs.at[1])

        plsc.subcore_barrier()
        pltpu.sync_copy(acc.at[pl.ds(s * _WR, _WR)],
                        out.at[c, pl.ds(s * _WR, _WR)])

    return pl.kernel(
        body,
        mesh=_mesh,
        out_type=jax.ShapeDtypeStruct((_NC, _ROWS_PAD, 128), jnp.float32),
        scratch_types=[
            pltpu.VMEM((_GRP, _CHUNK), jnp.int32),
            pltpu.VMEM((_GRP, _CHUNK), jnp.int32),
            pltpu.VMEM((2 * _CHUNK, 128), jnp.float32),
            pltpu.VMEM_SHARED((_ROWS_PAD, 128), jnp.float32),
            pltpu.SemaphoreType.DMA,
            pltpu.SemaphoreType.DMA,
        ],
    )


_segsum = _make_segsum()

def _make_degsum():
    n_chunks = _NCHUNKS // (_NC * _NS)
    grp = 16

    def body(e2, zeros, ones, out, dstb, rows1, acc, ssem):
        c = lax.axis_index("c")
        s = lax.axis_index("s")
        pltpu.sync_copy(zeros, acc.at[pl.ds(s * _WR, _WR)])
        pltpu.sync_copy(ones, rows1)
        plsc.subcore_barrier()

        def group_body(g, carry):
            base = c * (_NCHUNKS // 2) + s * n_chunks + g * grp
            pltpu.sync_copy(e2.at[1, pl.ds(base, grp)], dstb)
            sh = []
            for jj in range(grp):
                sh.append(pltpu.async_copy(
                    rows1, acc.at[dstb.at[jj]], ssem, add=True))
            for h in sh:
                h.wait()
            return carry
        lax.fori_loop(0, n_chunks // grp, group_body, 0)

        plsc.subcore_barrier()
        pltpu.sync_copy(acc.at[pl.ds(s * _WR, _WR)],
                        out.at[c, pl.ds(s * _WR, _WR)])

    return pl.kernel(
        body,
        mesh=_mesh,
        out_type=jax.ShapeDtypeStruct((_NC, _ROWS_PAD, 128), jnp.float32),
        scratch_types=[
            pltpu.VMEM((grp, _CHUNK), jnp.int32),
            pltpu.VMEM((_CHUNK, 128), jnp.float32),
            pltpu.VMEM_SHARED((_ROWS_PAD, 128), jnp.float32),
            pltpu.SemaphoreType.DMA,
        ],
    )


_degsum = _make_degsum()


_R = 1000
_G = N_NODES // _R


def _elu(v):
    return jnp.where(v > 0, v, jnp.exp(v) - 1.0)


def _mm_t(a, w):
    return lax.dot_general(a, w, (((1,), (1,)), ((), ())),
                           preferred_element_type=jnp.float32)


def _dis_of(cnt_ref):
    return lax.rsqrt(1.0 + cnt_ref[...])


def _edges_prep(ei3):
    blk = 128

    def body(e_ref, o_ref):
        i = pl.program_id(1)
        row = i * blk + lax.broadcasted_iota(jnp.int32, (blk, _CHUNK), 0)
        o_ref[...] = jnp.where(row < _NCH_IN, e_ref[...],
                               jnp.int32(N_NODES))

    return pl.pallas_call(
        body,
        grid=(2, _NCHUNKS // blk),
        in_specs=[pl.BlockSpec((1, blk, _CHUNK), lambda p, i: (p, i, 0))],
        out_specs=pl.BlockSpec((1, blk, _CHUNK), lambda p, i: (p, i, 0)),
        out_shape=jax.ShapeDtypeStruct((2, _NCHUNKS, _CHUNK), jnp.int32),
    )(ei3)


def _pad256(x):
    def body(x_ref, o_ref):
        o_ref[...] = jnp.concatenate(
            [x_ref[...], jnp.zeros((_R, 128), jnp.float32)], axis=1)

    return pl.pallas_call(
        body,
        grid=(_G,),
        in_specs=[pl.BlockSpec((_R, 128), lambda i: (i, 0))],
        out_specs=pl.BlockSpec((_R, 256), lambda i: (i, 0)),
        out_shape=jax.ShapeDtypeStruct((N_NODES, 256), jnp.float32),
    )(x)


def _take_left(h):
    def body(h_ref, o_ref):
        o_ref[...] = h_ref[:, :128]

    return pl.pallas_call(
        body,
        grid=(_G,),
        in_specs=[pl.BlockSpec((_R, 256), lambda i: (i, 0))],
        out_specs=pl.BlockSpec((_R, 128), lambda i: (i, 0)),
        out_shape=jax.ShapeDtypeStruct((N_NODES, 128), jnp.float32),
    )(h)


def _zscale(h, cnt):
    def body(h_ref, cnt_ref, z_ref):
        z = h_ref[...] * _dis_of(cnt_ref)
        z_ref[0] = z[:, :128]
        z_ref[1] = z[:, 128:]

    return pl.pallas_call(
        body,
        grid=(_G,),
        in_specs=[pl.BlockSpec((_R, 256), lambda i: (i, 0)),
                  pl.BlockSpec((_R, 1), lambda i: (i, 0))],
        out_specs=pl.BlockSpec((2, _R, 128), lambda i: (0, i, 0)),
        out_shape=jax.ShapeDtypeStruct((2, _ROWS_PAD, 128), jnp.float32),
    )(h, cnt)


def _combine(h, cnt, acc, w, b, ef):
    def body(h_ref, cnt_ref, a_ref, w_ref, b_ref, ef_ref, o_ref):
        dis = _dis_of(cnt_ref)
        hb = h_ref[...]
        acc2 = jnp.concatenate([a_ref[0], a_ref[1]], axis=1)
        agg = dis * (acc2 + dis * hb)
        hpre = ALPHA * hb + (1.0 - ALPHA) * agg
        v = _mm_t(hpre, w_ref[...]) + b_ref[...]
        o_ref[...] = jnp.where(ef_ref[...] > 0.5, _elu(v), v)

    return pl.pallas_call(
        body,
        grid=(_G,),
        in_specs=[pl.BlockSpec((_R, 256), lambda i: (i, 0)),
                  pl.BlockSpec((_R, 1), lambda i: (i, 0)),
                  pl.BlockSpec((2, _R, 128), lambda i: (0, i, 0)),
                  pl.BlockSpec((256, 256), lambda i: (0, 0)),
                  pl.BlockSpec((1, 256), lambda i: (0, 0)),
                  pl.BlockSpec((1, 1), lambda i: (0, 0))],
        out_specs=pl.BlockSpec((_R, 256), lambda i: (i, 0)),
        out_shape=jax.ShapeDtypeStruct((N_NODES, 256), jnp.float32),
    )(h, cnt, acc, w, b, ef)


def kernel(x, edge_index, W1, b1, W2, b2, W3, b3):
    ei3 = edge_index.astype(jnp.int32).reshape(2, _NCH_IN, _CHUNK)
    e2 = _edges_prep(ei3)
    zeros_blk = jnp.zeros((_WR, 128), jnp.float32)
    ones_sq = jnp.ones((_CHUNK, 128), jnp.float32)

    ws = jnp.stack([jnp.pad(W1, ((0, 0), (0, 128))), W2,
                    jnp.pad(W3, ((0, 128), (0, 0)))])
    bs = jnp.stack([b1, b2, jnp.pad(b3, (0, 128))]).reshape(3, 1, 256)
    efs = jnp.array([1.0, 1.0, 0.0], jnp.float32).reshape(3, 1, 1)

    acc_d = _degsum(e2, zeros_blk, ones_sq)
    cnt = acc_d[0][:, :1] + acc_d[1][:, :1]

    def layer(i, h):
        w = lax.dynamic_index_in_dim(ws, i, 0, keepdims=False)
        b = lax.dynamic_index_in_dim(bs, i, 0, keepdims=False)
        ef = lax.dynamic_index_in_dim(efs, i, 0, keepdims=False)
        zs = _zscale(h, cnt)
        acc = _segsum(zs, e2, zeros_blk)
        return _combine(h, cnt, acc, w, b, ef)

    h0 = _pad256(x)
    n_layers = 3 + (ei3[0, 0, 0] - ei3[0, 0, 0])
    h3 = lax.fori_loop(0, n_layers, layer, h0)
    return _take_left(h3)
